# baseline (speedup 1.0000x reference)
import functools

import jax
import jax.numpy as jnp
from jax import lax
from jax.experimental import pallas as pl
from jax.experimental.pallas import tpu as pltpu
from jax.experimental.pallas import tpu_sc as plsc

B = 16384
K = 8
D = 32
V = 1_000_000
TOT = B * K
NW = 32

CR_CH = 80
TR_CH = 4 * CR_CH
N_CH = V // TR_CH
TRIPS = -(-N_CH // NW)

CHUNK = 128
PER_W = TOT // NW
NCH = PER_W // CHUNK
NBUF = 8

_mesh = plsc.VectorSubcoreMesh(core_axis_name="c", subcore_axis_name="s")


@functools.partial(
    pl.kernel,
    mesh=_mesh,
    out_type=jax.ShapeDtypeStruct((V // 4, 128), jnp.float32),
    scratch_types=[
        pltpu.VMEM((TR_CH, D), jnp.float32),
        pltpu.VMEM((CR_CH, 128), jnp.float32),
        pltpu.SemaphoreType.DMA,
    ],
)
def _detile(table_hbm, cmp_hbm, in_v, out_v, sem):
    wid = lax.axis_index("s") * 2 + lax.axis_index("c")

    def chunk_body(j, _):
        c = wid + NW * j

        @pl.when(c < N_CH)
        def _():
            pltpu.async_copy(
                table_hbm.at[pl.ds(c * TR_CH, TR_CH)], in_v, sem
            ).wait()
            for lq in range(CR_CH):
                for g in range(8):
                    out_v[lq, pl.ds(g * 16, 16)] = in_v[
                        lq * 4 + g // 2, pl.ds((g % 2) * 16, 16)
                    ]
            pltpu.sync_copy(out_v, cmp_hbm.at[pl.ds(c * CR_CH, CR_CH)])

        return 0

    lax.fori_loop(0, TRIPS, chunk_body, 0)


@functools.partial(
    pl.kernel,
    mesh=_mesh,
    out_type=jax.ShapeDtypeStruct((TOT, D), jnp.float32),
    scratch_types=[
        pltpu.VMEM((NCH, CHUNK), jnp.int32),
        pltpu.VMEM((NBUF, CHUNK, D), jnp.float32),
    ]
    + [pltpu.SemaphoreType.DMA] * (2 * NBUF),
    compiler_params=pltpu.CompilerParams(use_tc_tiling_on_sc=False),
)
def _gather32(idx_hbm, table_hbm, out_hbm, idx_v, rows_v, *sems):
    gsems, osems = sems[:NBUF], sems[NBUF:]
    wid = lax.axis_index("s") * 2 + lax.axis_index("c")
    base = wid * PER_W
    pltpu.sync_copy(idx_hbm.at[pl.ds(wid * NCH, NCH)], idx_v)

    def gather(j, b):
        return pltpu.make_async_copy(
            table_hbm.at[idx_v.at[j]], rows_v.at[b], gsems[b])

    def writeback(j, b):
        return pltpu.make_async_copy(
            rows_v.at[b], out_hbm.at[pl.ds(base + j * CHUNK, CHUNK)],
            osems[b])

    for j in range(min(NBUF, NCH)):
        gather(j, j).start()
    for j in range(NCH):
        b = j % NBUF
        gather(j, b).wait()
        writeback(j, b).start()
        if j + NBUF < NCH:
            writeback(j, b).wait()
            gather(j + NBUF, b).start()
    for j in range(max(0, NCH - NBUF), NCH):
        writeback(j, j % NBUF).wait()


def kernel(point_feats_idx, values_weight):
    idx = point_feats_idx.astype(jnp.int32).reshape(TOT // 128, 128)
    compact = _detile(values_weight)
    flat = _gather32(idx, compact.reshape(V, D))
    return flat.reshape(B, K, D)
